# baseline (speedup 1.0000x reference)
import functools

import jax
import jax.numpy as jnp
from jax import lax
from jax.experimental import pallas as pl
from jax.experimental.pallas import tpu as pltpu
from jax.experimental.pallas import tpu_sc as plsc

_N_SC = 8192
_TC_ROWS = 2048
_SG = 32

_BIG = 1 << 30


def _argmax_body(a_ref, idx_ref):
    idx_ref[...] = jnp.argmax(a_ref[...], axis=1).astype(jnp.int32)


def _tc_argmax(frag_attr, n_tc):
    _, c = frag_attr.shape
    return pl.pallas_call(
        _argmax_body,
        grid=(n_tc // _TC_ROWS,),
        in_specs=[pl.BlockSpec((_TC_ROWS, c), lambda i: (i, 0))],
        out_specs=pl.BlockSpec((_TC_ROWS,), lambda i: (i,)),
        out_shape=jax.ShapeDtypeStruct((n_tc,), jnp.int32),
    )(frag_attr)


def _make_sc_argmax(n, c, row0, n_rows):
    info = plsc.get_sparse_core_info()
    nc, ns = info.num_cores, info.num_subcores
    nw = nc * ns
    r_per_w = n_rows // nw
    n_sg = r_per_w // _SG
    tail_off = c - 16
    n_full = -(-tail_off // 16)
    mesh = plsc.VectorSubcoreMesh(core_axis_name="c", subcore_axis_name="s")

    @functools.partial(
        pl.kernel,
        mesh=mesh,
        out_type=jax.ShapeDtypeStruct((n_rows,), jnp.int32),
        scratch_types=[
            pltpu.VMEM((_SG, c), jnp.float32),
            pltpu.VMEM((r_per_w,), jnp.int32),
        ],
    )
    def sc_argmax(frag_hbm, idx_hbm, stage_v, idx_v):
        wid = lax.axis_index("s") * nc + lax.axis_index("c")
        rbase = row0 + wid * r_per_w
        lane = jnp.arange(16, dtype=jnp.int32)

        def do_row(r16, dep, bestvec):
            neg_inf = jnp.full((16,), -jnp.inf, dtype=jnp.float32)
            zero_i = jnp.zeros((16,), dtype=jnp.int32)

            def slice_step(j, carry):
                cur, cidx = carry
                v = stage_v[r16, pl.ds(j * 16, 16)]
                pos = j * 16 + lane
                upd = v > cur
                return (
                    jnp.where(upd, v, cur),
                    jnp.where(upd, pos, cidx),
                )

            cur, cidx = lax.fori_loop(
                0, n_full, slice_step, (neg_inf, zero_i), unroll=8
            )
            v = stage_v[r16, pl.ds(tail_off, 16)]
            pos = tail_off + lane
            upd = v > cur
            cur = jnp.where(upd, v, cur)
            cidx = jnp.where(upd, pos, cidx)
            for sh in (8, 4, 2, 1):
                perm = lane ^ sh
                ocur = cur.at[perm].get(mode="promise_in_bounds")
                ocidx = cidx.at[perm].get(mode="promise_in_bounds")
                take_other = (ocur > cur) | ((ocur == cur) & (ocidx < cidx))
                cur = jnp.where(take_other, ocur, cur)
                cidx = jnp.where(take_other, ocidx, cidx)
            return jnp.where(lane == dep, cidx, bestvec)

        zero_i16 = jnp.zeros((16,), dtype=jnp.int32)
        for sg in range(n_sg):
            pltpu.sync_copy(
                frag_hbm.at[pl.ds(rbase + sg * _SG, _SG), :], stage_v
            )
            for g in range(_SG // 16):
                bestvec = lax.fori_loop(
                    0, 16, lambda r, bv, _g=g: do_row(_g * 16 + r, r, bv), zero_i16
                )
                idx_v[pl.ds(sg * _SG + g * 16, 16)] = bestvec

        pltpu.sync_copy(idx_v, idx_hbm.at[pl.ds(wid * r_per_w, r_per_w)])

    return sc_argmax


def _make_sc_gather(v, d, b):
    info = plsc.get_sparse_core_info()
    nc, ns = info.num_cores, info.num_subcores
    nw = nc * ns
    b_per_w = b // nw
    chunk = 128
    n_chunks = b_per_w // chunk
    mesh = plsc.VectorSubcoreMesh(core_axis_name="c", subcore_axis_name="s")

    @functools.partial(
        pl.kernel,
        mesh=mesh,
        out_type=jax.ShapeDtypeStruct((b, d), jnp.float32),
        scratch_types=[
            pltpu.VMEM((n_chunks, chunk), jnp.int32),
            pltpu.VMEM((b_per_w, d), jnp.float32),
            pltpu.SemaphoreType.DMA,
        ],
    )
    def gather_kernel(table_hbm, idx_hbm, out_hbm, idx_v, rows_v, sem):
        wid = lax.axis_index("s") * nc + lax.axis_index("c")
        base = wid * b_per_w
        for j in range(n_chunks):
            pltpu.sync_copy(idx_hbm.at[pl.ds(base + j * chunk, chunk)], idx_v.at[j])
        copies = [
            pltpu.async_copy(
                table_hbm.at[idx_v.at[j]],
                rows_v.at[pl.ds(j * chunk, chunk)],
                sem,
            )
            for j in range(n_chunks)
        ]
        for cp in copies:
            cp.wait()
        pltpu.sync_copy(rows_v, out_hbm.at[pl.ds(base, b_per_w)])

    return gather_kernel


def kernel(frag_attr, embedding_weight):
    n, c = frag_attr.shape
    v, d = embedding_weight.shape
    n_tc = n - _N_SC
    idx_tc = _tc_argmax(frag_attr, n_tc)
    idx_sc = _make_sc_argmax(n, c, n_tc, _N_SC)(frag_attr)
    idx = jnp.concatenate([idx_tc, idx_sc])
    return _make_sc_gather(v, d, n)(embedding_weight, idx)


# speedup vs baseline: 1.2145x; 1.2145x over previous
import functools

import jax
import jax.numpy as jnp
from jax import lax
from jax.experimental import pallas as pl
from jax.experimental.pallas import tpu as pltpu
from jax.experimental.pallas import tpu_sc as plsc

_ROWS = 2048


def _argmax_body(a_ref, idx_ref):
    idx_ref[...] = jnp.argmax(a_ref[...], axis=1).astype(jnp.int32)


def _make_sc_gather(v, d, b):
    info = plsc.get_sparse_core_info()
    nc, ns = info.num_cores, info.num_subcores
    nw = nc * ns
    b_per_w = b // nw
    chunk = 128
    n_chunks = b_per_w // chunk
    mesh = plsc.VectorSubcoreMesh(core_axis_name="c", subcore_axis_name="s")

    @functools.partial(
        pl.kernel,
        mesh=mesh,
        out_type=jax.ShapeDtypeStruct((b, d), jnp.float32),
        scratch_types=[
            pltpu.VMEM((n_chunks, chunk), jnp.int32),
            pltpu.VMEM((b_per_w, d), jnp.float32),
            pltpu.SemaphoreType.DMA,
        ],
    )
    def gather_kernel(table_hbm, idx_hbm, out_hbm, idx_v, rows_v, sem):
        wid = lax.axis_index("s") * nc + lax.axis_index("c")
        base = wid * b_per_w
        for j in range(n_chunks):
            pltpu.sync_copy(idx_hbm.at[pl.ds(base + j * chunk, chunk)], idx_v.at[j])
        copies = [
            pltpu.async_copy(
                table_hbm.at[idx_v.at[j]],
                rows_v.at[pl.ds(j * chunk, chunk)],
                sem,
            )
            for j in range(n_chunks)
        ]
        for cp in copies:
            cp.wait()
        pltpu.sync_copy(rows_v, out_hbm.at[pl.ds(base, b_per_w)])

    return gather_kernel


def kernel(frag_attr, embedding_weight):
    n, c = frag_attr.shape
    v, d = embedding_weight.shape
    idx = pl.pallas_call(
        _argmax_body,
        grid=(n // _ROWS,),
        in_specs=[pl.BlockSpec((_ROWS, c), lambda i: (i, 0))],
        out_specs=pl.BlockSpec((_ROWS,), lambda i: (i,)),
        out_shape=jax.ShapeDtypeStruct((n,), jnp.int32),
    )(frag_attr)
    return _make_sc_gather(v, d, n)(embedding_weight, idx)


# speedup vs baseline: 2.7038x; 2.2264x over previous
import functools

import jax
import jax.numpy as jnp
from jax import lax
from jax.experimental import pallas as pl
from jax.experimental.pallas import tpu as pltpu
from jax.experimental.pallas import tpu_sc as plsc

_COLS = 2048


def _argmax_t_body(a_ref, idx_ref):
    a = a_ref[...]
    m = jnp.max(a, axis=0)
    rows = lax.broadcasted_iota(jnp.int32, a.shape, 0)
    idx_ref[...] = jnp.min(jnp.where(a == m[None, :], rows, 1 << 30), axis=0)


def _make_sc_gather(v, d, b):
    info = plsc.get_sparse_core_info()
    nc, ns = info.num_cores, info.num_subcores
    nw = nc * ns
    b_per_w = b // nw
    chunk = 128
    n_chunks = b_per_w // chunk
    mesh = plsc.VectorSubcoreMesh(core_axis_name="c", subcore_axis_name="s")

    @functools.partial(
        pl.kernel,
        mesh=mesh,
        out_type=jax.ShapeDtypeStruct((b, d), jnp.float32),
        scratch_types=[
            pltpu.VMEM((n_chunks, chunk), jnp.int32),
            pltpu.VMEM((b_per_w, d), jnp.float32),
            pltpu.SemaphoreType.DMA,
        ],
    )
    def gather_kernel(table_hbm, idx_hbm, out_hbm, idx_v, rows_v, sem):
        wid = lax.axis_index("s") * nc + lax.axis_index("c")
        base = wid * b_per_w
        for j in range(n_chunks):
            pltpu.sync_copy(idx_hbm.at[pl.ds(base + j * chunk, chunk)], idx_v.at[j])
        copies = [
            pltpu.async_copy(
                table_hbm.at[idx_v.at[j]],
                rows_v.at[pl.ds(j * chunk, chunk)],
                sem,
            )
            for j in range(n_chunks)
        ]
        for cp in copies:
            cp.wait()
        pltpu.sync_copy(rows_v, out_hbm.at[pl.ds(base, b_per_w)])

    return gather_kernel


def kernel(frag_attr, embedding_weight):
    n, c = frag_attr.shape
    v, d = embedding_weight.shape
    ft = frag_attr.T
    idx = pl.pallas_call(
        _argmax_t_body,
        grid=(n // _COLS,),
        in_specs=[pl.BlockSpec((c, _COLS), lambda i: (0, i))],
        out_specs=pl.BlockSpec((_COLS,), lambda i: (i,)),
        out_shape=jax.ShapeDtypeStruct((n,), jnp.int32),
    )(ft)
    return _make_sc_gather(v, d, n)(embedding_weight, idx)
